# baseline (speedup 1.0000x reference)
import functools
import jax
import jax.numpy as jnp
from jax import lax
from jax.experimental import pallas as pl
from jax.experimental.pallas import tpu as pltpu
from jax.experimental.pallas import tpu_sc as plsc

D_M = 64
SCALE = 8.0
NW = 32
LANES = 16
NBUF = 4
AHEAD = 2


def _emb_call(R, C):
    RW = R // NW
    mesh = plsc.VectorSubcoreMesh(core_axis_name="c", subcore_axis_name="s")

    @functools.partial(
        pl.kernel,
        mesh=mesh,
        out_type=jax.ShapeDtypeStruct((R, C, D_M), jnp.float32),
        compiler_params=pltpu.CompilerParams(use_tc_tiling_on_sc=False),
        scratch_types=[
            pltpu.VMEM((RW, C), jnp.int32),
            pltpu.VMEM((NBUF, C, D_M), jnp.float32),
            pltpu.SemaphoreType.DMA((NBUF,)),
            pltpu.SemaphoreType.DMA((NBUF,)),
        ],
    )
    def body(idx_hbm, lut_hbm, out_hbm, idx_v, bufs, gsems, osems):
        wid = lax.axis_index("s") * 2 + lax.axis_index("c")
        rbase = wid * RW
        pltpu.sync_copy(idx_hbm.at[pl.ds(rbase, RW)], idx_v)

        for b in range(AHEAD):
            pltpu.async_copy(lut_hbm.at[idx_v.at[b]], bufs.at[b], gsems.at[b])

        def block(j0, carry):
            for b in range(NBUF):
                j = j0 + b
                jf = j + AHEAD
                bf = (b + AHEAD) % NBUF

                @pl.when(jf < RW)
                def _fire():
                    @pl.when(jf >= NBUF)
                    def _drain():
                        pltpu.make_async_copy(
                            bufs.at[bf],
                            out_hbm.at[rbase + jf - NBUF],
                            osems.at[bf],
                        ).wait()

                    pltpu.async_copy(
                        lut_hbm.at[idx_v.at[jf]], bufs.at[bf], gsems.at[bf]
                    )

                pltpu.make_async_copy(
                    lut_hbm.at[idx_v.at[j]], bufs.at[b], gsems.at[b]
                ).wait()

                def srow(t, c2):
                    for rr in range(2):
                        for q in range(D_M // LANES):
                            sl = pl.ds(q * LANES, LANES)
                            bufs[b, 2 * t + rr, sl] = bufs[b, 2 * t + rr, sl] * SCALE
                    return c2

                lax.fori_loop(0, C // 2, srow, 0, unroll=2)

                pltpu.async_copy(
                    bufs.at[b], out_hbm.at[rbase + j], osems.at[b]
                )
            return carry

        lax.fori_loop(0, RW // NBUF, lambda t, c: block(t * NBUF, c), 0)

        for b in range(NBUF):
            pltpu.make_async_copy(
                bufs.at[b], out_hbm.at[rbase + RW - NBUF + b], osems.at[b]
            ).wait()

    return body


def kernel(x, lut):
    xi = x.astype(jnp.int32)
    return _emb_call(x.shape[0], x.shape[1])(xi, lut)
